# baseline (speedup 1.0000x reference)
import functools

import jax
import jax.numpy as jnp
from jax import lax
from jax.experimental import pallas as pl
from jax.experimental.pallas import tpu as pltpu
from jax.experimental.pallas import tpu_sc as plsc

B = 1024
S = 20
E = 128
ROWS_PER_SEG = 12
NUM_SEG = B * (S + 1)
NC, NS = 2, 16
NW = NC * NS
SEG_PER_W = NUM_SEG // NW
CHUNK_SEG = 8
CHUNK_ROWS = CHUNK_SEG * ROWS_PER_SEG
NCHUNK = SEG_PER_W // CHUNK_SEG
NGROUP = E // 16

_mesh = plsc.VectorSubcoreMesh(core_axis_name="c", subcore_axis_name="s")


NBUF = 4


@functools.partial(
    pl.kernel,
    out_type=jax.ShapeDtypeStruct((NUM_SEG, E), jnp.float32),
    mesh=_mesh,
    scratch_types=[
        pltpu.VMEM((NCHUNK, CHUNK_ROWS), jnp.int32),
        [pltpu.VMEM((CHUNK_ROWS, E), jnp.float32) for _ in range(NBUF)],
        [pltpu.VMEM((CHUNK_SEG, E), jnp.float32) for _ in range(2)],
        [pltpu.SemaphoreType.DMA for _ in range(NBUF)],
        [pltpu.SemaphoreType.DMA for _ in range(2)],
    ],
)
def _embed_kernel(idx_hbm, table_hbm, out_hbm, idx_v, rows, outb, gsem, osem):
    wid = lax.axis_index("s") * NC + lax.axis_index("c")
    seg_base = wid * SEG_PER_W
    pltpu.sync_copy(idx_hbm.at[wid], idx_v)
    for b in range(NBUF - 1):
        pltpu.async_copy(table_hbm.at[idx_v.at[b]], rows[b], gsem[b])

    def ring_body(k, _):
        for b in range(NBUF):
            j = NBUF * k + b
            pltpu.make_async_copy(table_hbm.at[idx_v.at[j]], rows[b],
                                  gsem[b]).wait()
            nxt = j + NBUF - 1
            nb = (b + NBUF - 1) % NBUF

            @pl.when(nxt < NCHUNK)
            def _start_next():
                pltpu.async_copy(table_hbm.at[idx_v.at[nxt]], rows[nb],
                                 gsem[nb])

            ob = b % 2

            @pl.when(j >= 2)
            def _drain_out():
                pltpu.make_async_copy(
                    outb[ob],
                    out_hbm.at[pl.ds(seg_base, CHUNK_SEG)],
                    osem[ob]).wait()

            def seg_body(s, _, b=b, ob=ob):
                rbase = s * ROWS_PER_SEG
                for g in range(NGROUP):
                    sl = pl.ds(g * 16, 16)
                    acc = rows[b][rbase, sl]
                    for r in range(1, ROWS_PER_SEG):
                        acc = acc + rows[b][rbase + r, sl]
                    outb[ob][s, sl] = acc * (1.0 / ROWS_PER_SEG)
                return 0

            lax.fori_loop(0, CHUNK_SEG, seg_body, 0)
            pltpu.async_copy(
                outb[ob],
                out_hbm.at[pl.ds(seg_base + j * CHUNK_SEG, CHUNK_SEG)],
                osem[ob])
        return 0

    lax.fori_loop(0, NCHUNK // NBUF, ring_body, 0)
    for ob in range(2):
        pltpu.make_async_copy(outb[ob],
                              out_hbm.at[pl.ds(seg_base, CHUNK_SEG)],
                              osem[ob]).wait()


def kernel(sub_index, derived_sub_indices, action_mask, table):
    idx_all = jnp.concatenate(
        [sub_index.astype(jnp.int32).reshape(-1),
         derived_sub_indices.astype(jnp.int32).reshape(-1)])
    idx_all = idx_all.reshape(NW, NCHUNK, CHUNK_ROWS)
    out = _embed_kernel(idx_all, table)
    obs = out[:B].reshape(B, 1, E)
    act = out[B:].reshape(B, S, E)
    return (obs, act, action_mask)


# speedup vs baseline: 1.0267x; 1.0267x over previous
import functools

import jax
import jax.numpy as jnp
from jax import lax
from jax.experimental import pallas as pl
from jax.experimental.pallas import tpu as pltpu
from jax.experimental.pallas import tpu_sc as plsc

B = 1024
S = 20
E = 128
ROWS_PER_SEG = 12
NC, NS = 2, 16
NW = NC * NS
CHUNK_SEG = 8
CHUNK_ROWS = CHUNK_SEG * ROWS_PER_SEG
OBS_CHUNKS = B // (NW * CHUNK_SEG)
ACT_CHUNKS = B * S // (NW * CHUNK_SEG)
NCHUNK = OBS_CHUNKS + ACT_CHUNKS
NGROUP = E // 16
NBUF = 4

_mesh = plsc.VectorSubcoreMesh(core_axis_name="c", subcore_axis_name="s")


@functools.partial(
    pl.kernel,
    out_type=(jax.ShapeDtypeStruct((B, E), jnp.float32),
              jax.ShapeDtypeStruct((B * S, E), jnp.float32)),
    mesh=_mesh,
    scratch_types=[
        pltpu.VMEM((NCHUNK, CHUNK_ROWS), jnp.int32),
        [pltpu.VMEM((CHUNK_ROWS, E), jnp.float32) for _ in range(NBUF)],
        [pltpu.VMEM((CHUNK_SEG, E), jnp.float32) for _ in range(2)],
        [pltpu.SemaphoreType.DMA for _ in range(NBUF)],
        [pltpu.SemaphoreType.DMA for _ in range(2)],
    ],
)
def _embed_kernel(obs_idx_hbm, act_idx_hbm, table_hbm, obs_hbm, act_hbm,
                  idx_v, rows, outb, gsem, osem):
    wid = lax.axis_index("s") * NC + lax.axis_index("c")
    obs_base = wid * OBS_CHUNKS * CHUNK_SEG
    act_base = wid * ACT_CHUNKS * CHUNK_SEG
    pltpu.sync_copy(obs_idx_hbm.at[pl.ds(wid * OBS_CHUNKS, OBS_CHUNKS)],
                    idx_v.at[pl.ds(0, OBS_CHUNKS)])
    pltpu.sync_copy(act_idx_hbm.at[pl.ds(wid * ACT_CHUNKS, ACT_CHUNKS)],
                    idx_v.at[pl.ds(OBS_CHUNKS, ACT_CHUNKS)])
    for b in range(NBUF - 1):
        pltpu.async_copy(table_hbm.at[idx_v.at[b]], rows[b], gsem[b])

    def ring_body(k, _):
        for b in range(NBUF):
            j = NBUF * k + b
            pltpu.make_async_copy(table_hbm.at[idx_v.at[j]], rows[b],
                                  gsem[b]).wait()
            nxt = j + NBUF - 1
            nb = (b + NBUF - 1) % NBUF

            @pl.when(nxt < NCHUNK)
            def _start_next():
                pltpu.async_copy(table_hbm.at[idx_v.at[nxt]], rows[nb],
                                 gsem[nb])

            ob = b % 2

            @pl.when(j >= 2)
            def _drain_out():
                pltpu.make_async_copy(outb[ob],
                                      act_hbm.at[pl.ds(act_base, CHUNK_SEG)],
                                      osem[ob]).wait()

            def seg_body(s, _, b=b, ob=ob):
                rbase = s * ROWS_PER_SEG
                for g in range(NGROUP):
                    sl = pl.ds(g * 16, 16)
                    acc = rows[b][rbase, sl]
                    for r in range(1, ROWS_PER_SEG):
                        acc = acc + rows[b][rbase + r, sl]
                    outb[ob][s, sl] = acc * (1.0 / ROWS_PER_SEG)
                return 0

            lax.fori_loop(0, CHUNK_SEG, seg_body, 0)

            @pl.when(j < OBS_CHUNKS)
            def _flush_obs():
                pltpu.async_copy(
                    outb[ob],
                    obs_hbm.at[pl.ds(obs_base + j * CHUNK_SEG, CHUNK_SEG)],
                    osem[ob])

            @pl.when(j >= OBS_CHUNKS)
            def _flush_act():
                pltpu.async_copy(
                    outb[ob],
                    act_hbm.at[pl.ds(act_base + (j - OBS_CHUNKS) * CHUNK_SEG,
                                     CHUNK_SEG)],
                    osem[ob])
        return 0

    lax.fori_loop(0, NCHUNK // NBUF, ring_body, 0)
    for ob in range(2):
        pltpu.make_async_copy(outb[ob], act_hbm.at[pl.ds(act_base, CHUNK_SEG)],
                              osem[ob]).wait()


def kernel(sub_index, derived_sub_indices, action_mask, table):
    obs_idx = sub_index.astype(jnp.int32).reshape(B // CHUNK_SEG, CHUNK_ROWS)
    act_idx = derived_sub_indices.astype(jnp.int32).reshape(
        B * S // CHUNK_SEG, CHUNK_ROWS)
    obs, act = _embed_kernel(obs_idx, act_idx, table)
    return (obs.reshape(B, 1, E), act.reshape(B, S, E), action_mask)


# speedup vs baseline: 2.7849x; 2.7126x over previous
import functools

import jax
import jax.numpy as jnp
from jax import lax
from jax.experimental import pallas as pl
from jax.experimental.pallas import tpu as pltpu
from jax.experimental.pallas import tpu_sc as plsc

B = 1024
S = 20
E = 128
RPS = 12
NC, NS = 2, 16
NW = NC * NS
SEGW = B // NW
NGROUP = E // 16

_mesh = plsc.VectorSubcoreMesh(core_axis_name="c", subcore_axis_name="s")


@functools.partial(
    pl.kernel,
    out_type=(jax.ShapeDtypeStruct((B, E), jnp.float32),
              jax.ShapeDtypeStruct((S * B, E), jnp.float32)),
    mesh=_mesh,
    scratch_types=[
        pltpu.VMEM((RPS * SEGW,), jnp.int32),
        pltpu.VMEM((S * RPS * SEGW,), jnp.int32),
        [pltpu.VMEM((SEGW, E), jnp.float32) for _ in range(2 * RPS)],
        [pltpu.VMEM((SEGW, E), jnp.float32) for _ in range(2)],
        [pltpu.SemaphoreType.DMA for _ in range(2)],
        [pltpu.SemaphoreType.DMA for _ in range(2)],
    ],
)
def _embed_kernel(obs_idx_hbm, act_idx_hbm, table_hbm, obs_hbm, act_hbm,
                  idx_o, idx_a, bufs, outb, gsem, osem):
    wid = lax.axis_index("s") * NC + lax.axis_index("c")
    col = wid * SEGW

    pltpu.sync_copy(obs_idx_hbm.at[wid], idx_o)
    pltpu.sync_copy(act_idx_hbm.at[wid], idx_a)

    def issue_act(g, p):
        for i in range(RPS):
            pltpu.async_copy(
                table_hbm.at[idx_a.at[pl.ds((g * RPS + i) * SEGW, SEGW)]],
                bufs[RPS * p + i], gsem[p])

    def drain_g(p):
        for i in range(RPS):
            pltpu.make_async_copy(table_hbm.at[pl.ds(0, SEGW)],
                                  bufs[RPS * p + i], gsem[p]).wait()

    def wait_out(ob):
        pltpu.make_async_copy(table_hbm.at[pl.ds(0, SEGW)], outb[ob],
                              osem[ob]).wait()

    def reduce_store(p, ob, dst_ref, dst_row):
        def body(b, _):
            for gr in range(NGROUP):
                sl = pl.ds(gr * 16, 16)
                acc = bufs[RPS * p][b, sl]
                for i in range(1, RPS):
                    acc = acc + bufs[RPS * p + i][b, sl]
                outb[ob][b, sl] = acc * (1.0 / RPS)
            return 0

        lax.fori_loop(0, SEGW, body, 0)
        pltpu.async_copy(outb[ob], dst_ref.at[pl.ds(dst_row, SEGW)], osem[ob])

    for i in range(RPS):
        pltpu.async_copy(table_hbm.at[idx_o.at[pl.ds(i * SEGW, SEGW)]],
                         bufs[i], gsem[0])
    issue_act(0, 1)
    drain_g(0)
    reduce_store(0, 0, obs_hbm, col)

    def pair_body(k, _):
        g = 2 * k
        issue_act(g + 1, 0)
        drain_g(1)

        @pl.when(k > 0)
        def _w1():
            wait_out(1)

        reduce_store(1, 1, act_hbm, g * B + col)

        @pl.when(g + 2 < S)
        def _i2():
            issue_act(g + 2, 1)

        drain_g(0)
        wait_out(0)
        reduce_store(0, 0, act_hbm, (g + 1) * B + col)
        return 0

    lax.fori_loop(0, S // 2, pair_body, 0)
    wait_out(0)
    wait_out(1)


def kernel(sub_index, derived_sub_indices, action_mask, table):
    obs_t = jnp.transpose(sub_index.astype(jnp.int32),
                          (1, 3, 2, 0)).reshape(RPS, NW, SEGW)
    obs_w = jnp.transpose(obs_t, (1, 0, 2)).reshape(NW, RPS * SEGW)
    act_t = jnp.transpose(derived_sub_indices.astype(jnp.int32),
                          (1, 3, 2, 0)).reshape(S * RPS, NW, SEGW)
    act_w = jnp.transpose(act_t, (1, 0, 2)).reshape(NW, S * RPS * SEGW)
    obs, act = _embed_kernel(obs_w, act_w, table)
    obs = obs.reshape(B, 1, E)
    act = act.reshape(S, B, E).transpose(1, 0, 2)
    return (obs, act, action_mask)
